# baseline (speedup 1.0000x reference)
import functools

import jax
import jax.numpy as jnp
from jax import lax
from jax.experimental import pallas as pl
from jax.experimental.pallas import tpu as pltpu
from jax.experimental.pallas import tpu_sc as plsc

NUM_SPECIES = 4
B, A, AEV = 1024, 32, 384
N = B * A
D1, D2, D3 = 160, 128, 96
TOK_BLK = 4096
NBLK = N // TOK_BLK + NUM_SPECIES
C = NBLK * TOK_BLK
RR, RC = 256, 128

NCORE, NSUB = 2, 16
NW = NCORE * NSUB
TPW = N // NW
CHUNK = 128
NCH = TPW // CHUNK
TPC = N // NSUB
NCC = TPC // CHUNK


def _celu(x):
    return jnp.where(x > 0, x, jnp.exp(jnp.minimum(x, 0.0)) - 1.0)


def _route_kernel(s_ref, dst_ref, blk_ref):
    sp = s_ref[...]
    ci = lax.broadcasted_iota(jnp.int32, (RC, RC), 0)
    cj = lax.broadcasted_iota(jnp.int32, (RC, RC), 1)
    tri_col = (ci < cj).astype(jnp.float32)
    ri = lax.broadcasted_iota(jnp.int32, (RR, RR), 0)
    rj = lax.broadcasted_iota(jnp.int32, (RR, RR), 1)
    tri_row = (rj < ri).astype(jnp.float32)
    kb = lax.broadcasted_iota(jnp.int32, (1, RC), 1)
    dst = jnp.zeros((RR, RC), jnp.int32)
    sblk = jnp.zeros((1, RC), jnp.int32)
    vcnt = jnp.zeros((1, RC), jnp.int32)
    off = jnp.int32(0)
    cumblk = jnp.int32(0)
    for s in range(NUM_SPECIES):
        m = sp == s
        mf = m.astype(jnp.float32)
        prior = jnp.dot(mf, tri_col, preferred_element_type=jnp.float32)
        rowcnt = jnp.sum(mf, axis=1, keepdims=True)
        rowoff = jnp.dot(tri_row, rowcnt, preferred_element_type=jnp.float32)
        rank = (prior + rowoff).astype(jnp.int32)
        total = jnp.sum(mf).astype(jnp.int32)
        dst = jnp.where(m, off + rank, dst)
        nb = (total + TOK_BLK - 1) // TOK_BLK
        new_cumblk = cumblk + nb
        sblk = sblk + (kb >= new_cumblk).astype(jnp.int32)
        in_seg = (kb >= cumblk) & (kb < new_cumblk)
        v = jnp.clip(total - (kb - cumblk) * TOK_BLK, 0, TOK_BLK)
        vcnt = jnp.where(in_seg, v, vcnt)
        off = off + nb * TOK_BLK
        cumblk = new_cumblk
    dst_ref[...] = dst
    blk_ref[0, :] = jnp.minimum(sblk, NUM_SPECIES - 1)[0]
    blk_ref[1, :] = vcnt[0]


def _route(species_2d):
    return pl.pallas_call(
        _route_kernel,
        in_specs=[pl.BlockSpec((RR, RC), lambda: (0, 0))],
        out_specs=[pl.BlockSpec((RR, RC), lambda: (0, 0)),
                   pl.BlockSpec((2, RC), lambda: (0, 0))],
        out_shape=[jax.ShapeDtypeStruct((RR, RC), jnp.int32),
                   jax.ShapeDtypeStruct((2, RC), jnp.int32)],
    )(species_2d)


@functools.cache
def _mesh():
    return plsc.VectorSubcoreMesh(core_axis_name="c", subcore_axis_name="s",
                                  num_cores=NCORE, num_subcores=NSUB)


@functools.cache
def _sc_scatter_built():
    return functools.partial(
        pl.kernel,
        mesh=_mesh(),
        out_type=jax.ShapeDtypeStruct((C, AEV), jnp.float32),
        scratch_types=[
            pltpu.VMEM((NCH, CHUNK), jnp.int32),
            pltpu.VMEM((CHUNK, AEV), jnp.float32),
            pltpu.VMEM((CHUNK, AEV), jnp.float32),
            pltpu.SemaphoreType.DMA,
            pltpu.SemaphoreType.DMA,
        ],
    )(_sc_scatter_body)


def _sc_scatter_body(aev_hbm, dst_hbm, xs_hbm, idx_v, rows_a, rows_b,
                     rsem, wsem):
    cid = lax.axis_index("c")
    sid = lax.axis_index("s")
    wid = sid * NCORE + cid
    base = wid * TPW
    pltpu.sync_copy(dst_hbm.at[wid], idx_v)
    bufs = (rows_a, rows_b)
    reads = [None] * NCH
    writes = [None] * NCH
    for ch in range(2):
        reads[ch] = pltpu.make_async_copy(
            aev_hbm.at[pl.ds(base + ch * CHUNK, CHUNK)], bufs[ch], rsem)
        reads[ch].start()
    for ch in range(NCH):
        reads[ch].wait()
        writes[ch] = pltpu.make_async_copy(bufs[ch % 2],
                                           xs_hbm.at[idx_v.at[ch]], wsem)
        writes[ch].start()
        if ch + 2 < NCH:
            writes[ch].wait()
            reads[ch + 2] = pltpu.make_async_copy(
                aev_hbm.at[pl.ds(base + (ch + 2) * CHUNK, CHUNK)],
                bufs[ch % 2], rsem)
            reads[ch + 2].start()
    writes[NCH - 2].wait()
    writes[NCH - 1].wait()


def _mlp_kernel(sblk_ref, vcnt_ref, x_ref, w1_ref, b1_ref, w2_ref, b2_ref,
                w3_ref, b3_ref, w4_ref, b4_ref, o_ref):
    del sblk_ref
    x = x_ref[...]
    h = _celu(jnp.dot(x, w1_ref[0], preferred_element_type=jnp.float32)
              + b1_ref[0])
    h = _celu(jnp.dot(h, w2_ref[0], preferred_element_type=jnp.float32)
              + b2_ref[0])
    h = _celu(jnp.dot(h, w3_ref[0], preferred_element_type=jnp.float32)
              + b3_ref[0])
    e = (jnp.dot(h, w4_ref[0], preferred_element_type=jnp.float32)
         + b4_ref[0])
    valid = (lax.broadcasted_iota(jnp.int32, (TOK_BLK, 1), 0)
             < vcnt_ref[pl.program_id(0)])
    o_ref[...] = jnp.where(valid, e, 0.0)


def _mlp(sblk, vcnt, x_sorted, W1, b1, W2, b2, W3, b3, W4, b4):
    grid_spec = pltpu.PrefetchScalarGridSpec(
        num_scalar_prefetch=2,
        grid=(NBLK,),
        in_specs=[
            pl.BlockSpec((TOK_BLK, AEV), lambda b, sb, vc: (b, 0)),
            pl.BlockSpec((1, AEV, D1), lambda b, sb, vc: (sb[b], 0, 0)),
            pl.BlockSpec((1, 1, D1), lambda b, sb, vc: (sb[b], 0, 0)),
            pl.BlockSpec((1, D1, D2), lambda b, sb, vc: (sb[b], 0, 0)),
            pl.BlockSpec((1, 1, D2), lambda b, sb, vc: (sb[b], 0, 0)),
            pl.BlockSpec((1, D2, D3), lambda b, sb, vc: (sb[b], 0, 0)),
            pl.BlockSpec((1, 1, D3), lambda b, sb, vc: (sb[b], 0, 0)),
            pl.BlockSpec((1, D3, 1), lambda b, sb, vc: (sb[b], 0, 0)),
            pl.BlockSpec((1, 1, 1), lambda b, sb, vc: (sb[b], 0, 0)),
        ],
        out_specs=pl.BlockSpec((TOK_BLK, 1), lambda b, sb, vc: (b, 0)),
    )
    return pl.pallas_call(
        _mlp_kernel,
        grid_spec=grid_spec,
        out_shape=jax.ShapeDtypeStruct((C, 1), jnp.float32),
    )(sblk, vcnt, x_sorted,
      W1, b1.reshape(NUM_SPECIES, 1, D1),
      W2, b2.reshape(NUM_SPECIES, 1, D2),
      W3, b3.reshape(NUM_SPECIES, 1, D3),
      W4, b4.reshape(NUM_SPECIES, 1, 1))


@functools.cache
def _sc_combine_built():
    return functools.partial(
        pl.kernel,
        mesh=_mesh(),
        out_type=jax.ShapeDtypeStruct((B,), jnp.float32),
        scratch_types=[
            pltpu.VMEM((NCC, CHUNK), jnp.int32),
            pltpu.VMEM((NCC, CHUNK), jnp.int32),
            pltpu.VMEM((NCC, CHUNK), jnp.float32),
            pltpu.VMEM((B,), jnp.float32),
            pltpu.VMEM_SHARED((B,), jnp.float32),
            pltpu.SemaphoreType.DMA,
        ],
    )(_sc_combine_body)


def _sc_combine_body(e_hbm, dst_hbm, out_hbm, idx_v, mol_v, e_v, zero_v,
                     acc_sh, gsem):
    cid = lax.axis_index("c")
    sid = lax.axis_index("s")

    @pl.when(cid == 0)
    def _core0():
        base = sid * TPC
        pltpu.sync_copy(dst_hbm.at[sid], idx_v)
        gets = []
        for ch in range(NCC):
            cp = pltpu.make_async_copy(e_hbm.at[idx_v.at[ch]], e_v.at[ch],
                                       gsem)
            cp.start()
            gets.append(cp)
        for ch in range(NCC):
            for q in range(CHUNK // 16):
                t = lax.iota(jnp.int32, 16) + (base + ch * CHUNK + q * 16)
                mol_v[ch, pl.ds(q * 16, 16)] = jnp.right_shift(t, 5)

        @pl.when(sid == 0)
        def _init():
            for q in range(B // 16):
                zero_v[pl.ds(q * 16, 16)] = jnp.zeros((16,), jnp.float32)
            pltpu.sync_copy(zero_v, acc_sh)

        for cp in gets:
            cp.wait()
        plsc.subcore_barrier()
        for ch in range(NCC):
            pltpu.sync_copy(e_v.at[ch], acc_sh.at[mol_v.at[ch]], add=True)
        plsc.subcore_barrier()

        @pl.when(sid == 0)
        def _emit():
            pltpu.sync_copy(acc_sh, out_hbm)


def kernel(species, aev, W1, b1, W2, b2, W3, b3, W4, b4):
    species_2d = species.reshape(RR, RC)
    aev_flat = aev.reshape(N, AEV)
    dst, blk = _route(species_2d)
    sblk = blk[0, :NBLK]
    vcnt = blk[1, :NBLK]
    x_sorted = _sc_scatter_built()(aev_flat, dst.reshape(NW, NCH, CHUNK))
    e_pad = _mlp(sblk, vcnt, x_sorted, W1, b1, W2, b2, W3, b3, W4, b4)
    return _sc_combine_built()(e_pad.reshape(C),
                               dst.reshape(NSUB, NCC, CHUNK))


# speedup vs baseline: 1.0301x; 1.0301x over previous
import functools

import jax
import jax.numpy as jnp
from jax import lax
from jax.experimental import pallas as pl
from jax.experimental.pallas import tpu as pltpu
from jax.experimental.pallas import tpu_sc as plsc

NUM_SPECIES = 4
B, A, AEV = 1024, 32, 384
N = B * A
D1, D2, D3 = 160, 128, 96
TOK_BLK = 2048
NBLK = N // TOK_BLK + NUM_SPECIES
C = NBLK * TOK_BLK
RR, RC = 256, 128

NCORE, NSUB = 2, 16
NW = NCORE * NSUB
TPW = N // NW
CHUNK = 128
NCH = TPW // CHUNK
TPC = N // NSUB
NCC = TPC // CHUNK


def _celu(x):
    return jnp.where(x > 0, x, jnp.exp(jnp.minimum(x, 0.0)) - 1.0)


def _route_kernel(s_ref, dst_ref, blk_ref):
    sp = s_ref[...]
    ci = lax.broadcasted_iota(jnp.int32, (RC, RC), 0)
    cj = lax.broadcasted_iota(jnp.int32, (RC, RC), 1)
    tri_col = (ci < cj).astype(jnp.float32)
    ri = lax.broadcasted_iota(jnp.int32, (RR, RR), 0)
    rj = lax.broadcasted_iota(jnp.int32, (RR, RR), 1)
    tri_row = (rj < ri).astype(jnp.float32)
    kb = lax.broadcasted_iota(jnp.int32, (1, RC), 1)
    dst = jnp.zeros((RR, RC), jnp.int32)
    sblk = jnp.zeros((1, RC), jnp.int32)
    vcnt = jnp.zeros((1, RC), jnp.int32)
    off = jnp.int32(0)
    cumblk = jnp.int32(0)
    for s in range(NUM_SPECIES):
        m = sp == s
        mf = m.astype(jnp.float32)
        prior = jnp.dot(mf, tri_col, preferred_element_type=jnp.float32)
        rowcnt = jnp.sum(mf, axis=1, keepdims=True)
        rowoff = jnp.dot(tri_row, rowcnt, preferred_element_type=jnp.float32)
        rank = (prior + rowoff).astype(jnp.int32)
        total = jnp.sum(mf).astype(jnp.int32)
        dst = jnp.where(m, off + rank, dst)
        nb = (total + TOK_BLK - 1) // TOK_BLK
        new_cumblk = cumblk + nb
        sblk = sblk + (kb >= new_cumblk).astype(jnp.int32)
        in_seg = (kb >= cumblk) & (kb < new_cumblk)
        v = jnp.clip(total - (kb - cumblk) * TOK_BLK, 0, TOK_BLK)
        vcnt = jnp.where(in_seg, v, vcnt)
        off = off + nb * TOK_BLK
        cumblk = new_cumblk
    dst_ref[...] = dst
    blk_ref[0, :] = jnp.minimum(sblk, NUM_SPECIES - 1)[0]
    blk_ref[1, :] = vcnt[0]


def _route(species_2d):
    return pl.pallas_call(
        _route_kernel,
        in_specs=[pl.BlockSpec((RR, RC), lambda: (0, 0))],
        out_specs=[pl.BlockSpec((RR, RC), lambda: (0, 0)),
                   pl.BlockSpec((2, RC), lambda: (0, 0))],
        out_shape=[jax.ShapeDtypeStruct((RR, RC), jnp.int32),
                   jax.ShapeDtypeStruct((2, RC), jnp.int32)],
    )(species_2d)


@functools.cache
def _mesh():
    return plsc.VectorSubcoreMesh(core_axis_name="c", subcore_axis_name="s",
                                  num_cores=NCORE, num_subcores=NSUB)


@functools.cache
def _sc_scatter_built():
    return functools.partial(
        pl.kernel,
        mesh=_mesh(),
        out_type=jax.ShapeDtypeStruct((C, AEV), jnp.float32),
        scratch_types=[
            pltpu.VMEM((NCH, CHUNK), jnp.int32),
            pltpu.VMEM((CHUNK, AEV), jnp.float32),
            pltpu.VMEM((CHUNK, AEV), jnp.float32),
            pltpu.SemaphoreType.DMA,
            pltpu.SemaphoreType.DMA,
        ],
    )(_sc_scatter_body)


def _sc_scatter_body(aev_hbm, dst_hbm, xs_hbm, idx_v, rows_a, rows_b,
                     rsem, wsem):
    cid = lax.axis_index("c")
    sid = lax.axis_index("s")
    wid = sid * NCORE + cid
    base = wid * TPW
    pltpu.sync_copy(dst_hbm.at[wid], idx_v)
    bufs = (rows_a, rows_b)
    reads = [None] * NCH
    writes = [None] * NCH
    for ch in range(2):
        reads[ch] = pltpu.make_async_copy(
            aev_hbm.at[pl.ds(base + ch * CHUNK, CHUNK)], bufs[ch], rsem)
        reads[ch].start()
    for ch in range(NCH):
        reads[ch].wait()
        writes[ch] = pltpu.make_async_copy(bufs[ch % 2],
                                           xs_hbm.at[idx_v.at[ch]], wsem)
        writes[ch].start()
        if ch + 2 < NCH:
            writes[ch].wait()
            reads[ch + 2] = pltpu.make_async_copy(
                aev_hbm.at[pl.ds(base + (ch + 2) * CHUNK, CHUNK)],
                bufs[ch % 2], rsem)
            reads[ch + 2].start()
    writes[NCH - 2].wait()
    writes[NCH - 1].wait()


def _mlp_kernel(sblk_ref, vcnt_ref, x_ref, w1_ref, b1_ref, w2_ref, b2_ref,
                w3_ref, b3_ref, w4_ref, b4_ref, o_ref):
    del sblk_ref
    x = x_ref[...].astype(jnp.bfloat16)
    h = _celu(jnp.dot(x, w1_ref[0].astype(jnp.bfloat16),
                      preferred_element_type=jnp.float32) + b1_ref[0])
    h = _celu(jnp.dot(h.astype(jnp.bfloat16), w2_ref[0].astype(jnp.bfloat16),
                      preferred_element_type=jnp.float32) + b2_ref[0])
    h = _celu(jnp.dot(h.astype(jnp.bfloat16), w3_ref[0].astype(jnp.bfloat16),
                      preferred_element_type=jnp.float32) + b3_ref[0])
    e = (jnp.dot(h.astype(jnp.bfloat16), w4_ref[0].astype(jnp.bfloat16),
                 preferred_element_type=jnp.float32)
         + b4_ref[0])
    valid = (lax.broadcasted_iota(jnp.int32, (TOK_BLK, 1), 0)
             < vcnt_ref[pl.program_id(0)])
    o_ref[...] = jnp.where(valid, e, 0.0)


def _mlp(sblk, vcnt, x_sorted, W1, b1, W2, b2, W3, b3, W4, b4):
    grid_spec = pltpu.PrefetchScalarGridSpec(
        num_scalar_prefetch=2,
        grid=(NBLK,),
        in_specs=[
            pl.BlockSpec((TOK_BLK, AEV), lambda b, sb, vc: (b, 0)),
            pl.BlockSpec((1, AEV, D1), lambda b, sb, vc: (sb[b], 0, 0)),
            pl.BlockSpec((1, 1, D1), lambda b, sb, vc: (sb[b], 0, 0)),
            pl.BlockSpec((1, D1, D2), lambda b, sb, vc: (sb[b], 0, 0)),
            pl.BlockSpec((1, 1, D2), lambda b, sb, vc: (sb[b], 0, 0)),
            pl.BlockSpec((1, D2, D3), lambda b, sb, vc: (sb[b], 0, 0)),
            pl.BlockSpec((1, 1, D3), lambda b, sb, vc: (sb[b], 0, 0)),
            pl.BlockSpec((1, D3, 1), lambda b, sb, vc: (sb[b], 0, 0)),
            pl.BlockSpec((1, 1, 1), lambda b, sb, vc: (sb[b], 0, 0)),
        ],
        out_specs=pl.BlockSpec((TOK_BLK, 1), lambda b, sb, vc: (b, 0)),
    )
    return pl.pallas_call(
        _mlp_kernel,
        grid_spec=grid_spec,
        out_shape=jax.ShapeDtypeStruct((C, 1), jnp.float32),
    )(sblk, vcnt, x_sorted,
      W1, b1.reshape(NUM_SPECIES, 1, D1),
      W2, b2.reshape(NUM_SPECIES, 1, D2),
      W3, b3.reshape(NUM_SPECIES, 1, D3),
      W4, b4.reshape(NUM_SPECIES, 1, 1))


@functools.cache
def _sc_combine_built():
    return functools.partial(
        pl.kernel,
        mesh=_mesh(),
        out_type=jax.ShapeDtypeStruct((B,), jnp.float32),
        scratch_types=[
            pltpu.VMEM((NCC, CHUNK), jnp.int32),
            pltpu.VMEM((NCC, CHUNK), jnp.int32),
            pltpu.VMEM((NCC, CHUNK), jnp.float32),
            pltpu.VMEM((B,), jnp.float32),
            pltpu.VMEM_SHARED((B,), jnp.float32),
            pltpu.SemaphoreType.DMA,
        ],
    )(_sc_combine_body)


def _sc_combine_body(e_hbm, dst_hbm, out_hbm, idx_v, mol_v, e_v, zero_v,
                     acc_sh, gsem):
    cid = lax.axis_index("c")
    sid = lax.axis_index("s")

    @pl.when(cid == 0)
    def _core0():
        base = sid * TPC
        pltpu.sync_copy(dst_hbm.at[sid], idx_v)
        gets = []
        for ch in range(NCC):
            cp = pltpu.make_async_copy(e_hbm.at[idx_v.at[ch]], e_v.at[ch],
                                       gsem)
            cp.start()
            gets.append(cp)
        for ch in range(NCC):
            for q in range(CHUNK // 16):
                t = lax.iota(jnp.int32, 16) + (base + ch * CHUNK + q * 16)
                mol_v[ch, pl.ds(q * 16, 16)] = jnp.right_shift(t, 5)

        @pl.when(sid == 0)
        def _init():
            for q in range(B // 16):
                zero_v[pl.ds(q * 16, 16)] = jnp.zeros((16,), jnp.float32)
            pltpu.sync_copy(zero_v, acc_sh)

        for cp in gets:
            cp.wait()
        plsc.subcore_barrier()
        for ch in range(NCC):
            pltpu.sync_copy(e_v.at[ch], acc_sh.at[mol_v.at[ch]], add=True)
        plsc.subcore_barrier()

        @pl.when(sid == 0)
        def _emit():
            pltpu.sync_copy(acc_sh, out_hbm)


def kernel(species, aev, W1, b1, W2, b2, W3, b3, W4, b4):
    species_2d = species.reshape(RR, RC)
    aev_flat = aev.reshape(N, AEV)
    dst, blk = _route(species_2d)
    sblk = blk[0, :NBLK]
    vcnt = blk[1, :NBLK]
    x_sorted = _sc_scatter_built()(aev_flat, dst.reshape(NW, NCH, CHUNK))
    e_pad = _mlp(sblk, vcnt, x_sorted, W1, b1, W2, b2, W3, b3, W4, b4)
    return _sc_combine_built()(e_pad.reshape(C),
                               dst.reshape(NSUB, NCC, CHUNK))


# speedup vs baseline: 1.0498x; 1.0192x over previous
import functools

import jax
import jax.numpy as jnp
from jax import lax
from jax.experimental import pallas as pl
from jax.experimental.pallas import tpu as pltpu
from jax.experimental.pallas import tpu_sc as plsc

NUM_SPECIES = 4
B, A, AEV = 1024, 32, 384
N = B * A
D1, D2, D3 = 160, 128, 96
TOK_BLK = 2048
NBLK = N // TOK_BLK + NUM_SPECIES
C = NBLK * TOK_BLK
RR, RC = 256, 128

NCORE, NSUB = 2, 16
NW = NCORE * NSUB
TPW = N // NW
CHUNK = 128
NCH = TPW // CHUNK
TPC = N // NSUB
NCC = TPC // CHUNK


def _celu(x):
    return jnp.where(x > 0, x, jnp.exp(x) - 1.0)


def _route_kernel(s_ref, dst_ref, blk_ref):
    sp = s_ref[...]
    ci = lax.broadcasted_iota(jnp.int32, (RC, RC), 0)
    cj = lax.broadcasted_iota(jnp.int32, (RC, RC), 1)
    tri_col = (ci < cj).astype(jnp.float32)
    ri = lax.broadcasted_iota(jnp.int32, (RR, RR), 0)
    rj = lax.broadcasted_iota(jnp.int32, (RR, RR), 1)
    tri_row = (rj < ri).astype(jnp.float32)
    kb = lax.broadcasted_iota(jnp.int32, (1, RC), 1)
    dst = jnp.zeros((RR, RC), jnp.int32)
    sblk = jnp.zeros((1, RC), jnp.int32)
    vcnt = jnp.zeros((1, RC), jnp.int32)
    off = jnp.int32(0)
    cumblk = jnp.int32(0)
    for s in range(NUM_SPECIES):
        m = sp == s
        mf = m.astype(jnp.float32)
        prior = jnp.dot(mf, tri_col, preferred_element_type=jnp.float32)
        rowcnt = jnp.sum(mf, axis=1, keepdims=True)
        rowoff = jnp.dot(tri_row, rowcnt, preferred_element_type=jnp.float32)
        rank = (prior + rowoff).astype(jnp.int32)
        total = jnp.sum(mf).astype(jnp.int32)
        dst = jnp.where(m, off + rank, dst)
        nb = (total + TOK_BLK - 1) // TOK_BLK
        new_cumblk = cumblk + nb
        sblk = sblk + (kb >= new_cumblk).astype(jnp.int32)
        in_seg = (kb >= cumblk) & (kb < new_cumblk)
        v = jnp.clip(total - (kb - cumblk) * TOK_BLK, 0, TOK_BLK)
        vcnt = jnp.where(in_seg, v, vcnt)
        off = off + nb * TOK_BLK
        cumblk = new_cumblk
    dst_ref[...] = dst
    blk_ref[0, :] = jnp.minimum(sblk, NUM_SPECIES - 1)[0]
    blk_ref[1, :] = vcnt[0]


def _route(species_2d):
    return pl.pallas_call(
        _route_kernel,
        in_specs=[pl.BlockSpec((RR, RC), lambda: (0, 0))],
        out_specs=[pl.BlockSpec((RR, RC), lambda: (0, 0)),
                   pl.BlockSpec((2, RC), lambda: (0, 0))],
        out_shape=[jax.ShapeDtypeStruct((RR, RC), jnp.int32),
                   jax.ShapeDtypeStruct((2, RC), jnp.int32)],
    )(species_2d)


@functools.cache
def _mesh():
    return plsc.VectorSubcoreMesh(core_axis_name="c", subcore_axis_name="s",
                                  num_cores=NCORE, num_subcores=NSUB)


@functools.cache
def _sc_scatter_built():
    return functools.partial(
        pl.kernel,
        mesh=_mesh(),
        out_type=jax.ShapeDtypeStruct((C, AEV), jnp.float32),
        scratch_types=[
            pltpu.VMEM((NCH, CHUNK), jnp.int32),
            pltpu.VMEM((CHUNK, AEV), jnp.float32),
            pltpu.VMEM((CHUNK, AEV), jnp.float32),
            pltpu.SemaphoreType.DMA,
            pltpu.SemaphoreType.DMA,
        ],
    )(_sc_scatter_body)


def _sc_scatter_body(aev_hbm, dst_hbm, xs_hbm, idx_v, rows_a, rows_b,
                     rsem, wsem):
    cid = lax.axis_index("c")
    sid = lax.axis_index("s")
    wid = sid * NCORE + cid
    base = wid * TPW
    pltpu.sync_copy(dst_hbm.at[wid], idx_v)
    bufs = (rows_a, rows_b)
    reads = [None] * NCH
    writes = [None] * NCH
    for ch in range(2):
        reads[ch] = pltpu.make_async_copy(
            aev_hbm.at[pl.ds(base + ch * CHUNK, CHUNK)], bufs[ch], rsem)
        reads[ch].start()
    for ch in range(NCH):
        reads[ch].wait()
        writes[ch] = pltpu.make_async_copy(bufs[ch % 2],
                                           xs_hbm.at[idx_v.at[ch]], wsem)
        writes[ch].start()
        if ch + 2 < NCH:
            writes[ch].wait()
            reads[ch + 2] = pltpu.make_async_copy(
                aev_hbm.at[pl.ds(base + (ch + 2) * CHUNK, CHUNK)],
                bufs[ch % 2], rsem)
            reads[ch + 2].start()
    writes[NCH - 2].wait()
    writes[NCH - 1].wait()


def _mlp_kernel(sblk_ref, vcnt_ref, x_ref, w1_ref, b1_ref, w2_ref, b2_ref,
                w3_ref, b3_ref, w4_ref, b4_ref, o_ref):
    del sblk_ref
    x = x_ref[...]
    h = _celu(jnp.dot(x, w1_ref[0], preferred_element_type=jnp.float32)
              + b1_ref[0])
    h = _celu(jnp.dot(h, w2_ref[0], preferred_element_type=jnp.float32)
              + b2_ref[0])
    h = _celu(jnp.dot(h, w3_ref[0], preferred_element_type=jnp.float32)
              + b3_ref[0])
    e = (jnp.dot(h, w4_ref[0], preferred_element_type=jnp.float32)
         + b4_ref[0])
    valid = (lax.broadcasted_iota(jnp.int32, (TOK_BLK, 1), 0)
             < vcnt_ref[pl.program_id(0)])
    o_ref[...] = jnp.where(valid, e, 0.0)


def _mlp(sblk, vcnt, x_sorted, W1, b1, W2, b2, W3, b3, W4, b4):
    grid_spec = pltpu.PrefetchScalarGridSpec(
        num_scalar_prefetch=2,
        grid=(NBLK,),
        in_specs=[
            pl.BlockSpec((TOK_BLK, AEV), lambda b, sb, vc: (b, 0)),
            pl.BlockSpec((1, AEV, D1), lambda b, sb, vc: (sb[b], 0, 0)),
            pl.BlockSpec((1, 1, D1), lambda b, sb, vc: (sb[b], 0, 0)),
            pl.BlockSpec((1, D1, D2), lambda b, sb, vc: (sb[b], 0, 0)),
            pl.BlockSpec((1, 1, D2), lambda b, sb, vc: (sb[b], 0, 0)),
            pl.BlockSpec((1, D2, D3), lambda b, sb, vc: (sb[b], 0, 0)),
            pl.BlockSpec((1, 1, D3), lambda b, sb, vc: (sb[b], 0, 0)),
            pl.BlockSpec((1, D3, 1), lambda b, sb, vc: (sb[b], 0, 0)),
            pl.BlockSpec((1, 1, 1), lambda b, sb, vc: (sb[b], 0, 0)),
        ],
        out_specs=pl.BlockSpec((TOK_BLK, 1), lambda b, sb, vc: (b, 0)),
    )
    return pl.pallas_call(
        _mlp_kernel,
        grid_spec=grid_spec,
        out_shape=jax.ShapeDtypeStruct((C, 1), jnp.float32),
    )(sblk, vcnt, x_sorted,
      W1, b1.reshape(NUM_SPECIES, 1, D1),
      W2, b2.reshape(NUM_SPECIES, 1, D2),
      W3, b3.reshape(NUM_SPECIES, 1, D3),
      W4, b4.reshape(NUM_SPECIES, 1, 1))


@functools.cache
def _sc_combine_built():
    return functools.partial(
        pl.kernel,
        mesh=_mesh(),
        out_type=jax.ShapeDtypeStruct((B,), jnp.float32),
        scratch_types=[
            pltpu.VMEM((NCC, CHUNK), jnp.int32),
            pltpu.VMEM((NCC, CHUNK), jnp.int32),
            pltpu.VMEM((NCC, CHUNK), jnp.float32),
            pltpu.VMEM((B,), jnp.float32),
            pltpu.VMEM_SHARED((B,), jnp.float32),
            pltpu.SemaphoreType.DMA,
        ],
    )(_sc_combine_body)


def _sc_combine_body(e_hbm, dst_hbm, out_hbm, idx_v, mol_v, e_v, zero_v,
                     acc_sh, gsem):
    cid = lax.axis_index("c")
    sid = lax.axis_index("s")

    @pl.when(cid == 0)
    def _core0():
        base = sid * TPC
        pltpu.sync_copy(dst_hbm.at[sid], idx_v)
        gets = []
        for ch in range(NCC):
            cp = pltpu.make_async_copy(e_hbm.at[idx_v.at[ch]], e_v.at[ch],
                                       gsem)
            cp.start()
            gets.append(cp)
        for ch in range(NCC):
            for q in range(CHUNK // 16):
                t = lax.iota(jnp.int32, 16) + (base + ch * CHUNK + q * 16)
                mol_v[ch, pl.ds(q * 16, 16)] = jnp.right_shift(t, 5)

        @pl.when(sid == 0)
        def _init():
            for q in range(B // 16):
                zero_v[pl.ds(q * 16, 16)] = jnp.zeros((16,), jnp.float32)
            pltpu.sync_copy(zero_v, acc_sh)

        for cp in gets:
            cp.wait()
        plsc.subcore_barrier()
        for ch in range(NCC):
            pltpu.sync_copy(e_v.at[ch], acc_sh.at[mol_v.at[ch]], add=True)
        plsc.subcore_barrier()

        @pl.when(sid == 0)
        def _emit():
            pltpu.sync_copy(acc_sh, out_hbm)


def kernel(species, aev, W1, b1, W2, b2, W3, b3, W4, b4):
    species_2d = species.reshape(RR, RC)
    aev_flat = aev.reshape(N, AEV)
    dst, blk = _route(species_2d)
    sblk = blk[0, :NBLK]
    vcnt = blk[1, :NBLK]
    x_sorted = _sc_scatter_built()(aev_flat, dst.reshape(NW, NCH, CHUNK))
    e_pad = _mlp(sblk, vcnt, x_sorted, W1, b1, W2, b2, W3, b3, W4, b4)
    return _sc_combine_built()(e_pad.reshape(C),
                               dst.reshape(NSUB, NCC, CHUNK))
